# baseline (speedup 1.0000x reference)
import jax
import jax.numpy as jnp
from jax import lax
from jax.experimental import pallas as pl
from jax.experimental.pallas import tpu as pltpu
from jax.experimental.pallas import tpu_sc as plsc

H = 512
W = 512
NPATH = 7
NB = 11
LANES = 16
NPROB = 8 * 2 * NPATH
BROWS = 24
DOFF = 3
BIG = 1e30


def _gm_body(x_ref, out_ref):
    xb = x_ref[0]
    gray = 0.2989 * xb[0] + 0.587 * xb[1] + 0.114 * xb[2]
    zr = jnp.zeros((1, W), jnp.float32)
    zc = jnp.zeros((H, 1), jnp.float32)
    up = jnp.concatenate([gray[1:], zr], axis=0)
    dn = jnp.concatenate([zr, gray[:-1]], axis=0)
    s = dn + 2.0 * gray + up
    gx = (jnp.concatenate([s[:, 1:], zc], axis=1)
          - jnp.concatenate([zc, s[:, :-1]], axis=1))
    d = (jnp.concatenate([gray[:, 1:], zc], axis=1) + 2.0 * gray
         + jnp.concatenate([zc, gray[:, :-1]], axis=1))
    gy = (jnp.concatenate([d[1:], zr], axis=0)
          - jnp.concatenate([zr, d[:-1]], axis=0))
    gm = jnp.sqrt(gx * gx + gy * gy + 1e-8)
    gmt = gm.T
    for p in range(NPATH):
        c0 = 64 * p + 56
        out_ref[0, 0, p] = gm[c0:c0 + BROWS, :]
        out_ref[0, 1, p] = gmt[c0:c0 + BROWS, :]


def _gradient_map_pallas(x):
    return pl.pallas_call(
        _gm_body,
        grid=(8,),
        in_specs=[pl.BlockSpec((1, 3, H, W), lambda b: (b, 0, 0, 0))],
        out_specs=pl.BlockSpec((1, 2, NPATH, BROWS, W),
                               lambda b: (b, 0, 0, 0, 0)),
        out_shape=jax.ShapeDtypeStruct((8, 2, NPATH, BROWS, W), jnp.float32),
    )(x)


NWAY = 4


def _sc_body(g2_hbm, out_hbm, band0, band1, band2, band3, dec, pos):
    bands = [band0, band1, band2, band3]
    wid = lax.axis_index("s") * 2 + lax.axis_index("c")
    iota = lax.iota(jnp.int32, LANES)
    lane_ok = iota < NB
    iota_d = iota + DOFF
    jl = jnp.maximum(iota - 1, 0)
    jr = jnp.minimum(iota + 1, LANES - 1)
    zero16 = jnp.zeros((LANES,), jnp.int32)
    lane0 = iota == 0

    pids = [wid + 32 * k for k in range(NWAY)]
    r0s = []
    for k in range(NWAY):
        pid = jnp.minimum(pids[k], NPROB - 1)
        b = pid // 14
        t = pid - 14 * b
        orient = jnp.where(t < NPATH, 1, 0)
        p = lax.rem(t, NPATH)
        r0s.append(64 * p + 59)
        pltpu.sync_copy(g2_hbm.at[b, orient, p, :, :], bands[k])

    def g_load(k, l):
        return plsc.load_gather(bands[k], [iota_d, l + zero16])

    c0s = tuple(jnp.where(lane_ok, -g_load(k, 0), BIG) for k in range(NWAY))

    def fwd(l, cs):
        cns = []
        for k in range(NWAY):
            c = cs[k]
            lv = c.at[jl].get(mode="promise_in_bounds")
            rv = c.at[jr].get(mode="promise_in_bounds")
            isl = jnp.logical_and(lv <= c, lv <= rv)
            isc = jnp.logical_and(jnp.logical_not(isl), c <= rv)
            jp = jnp.where(isl, jl, jnp.where(isc, iota, jr))
            m = jnp.where(isl, lv, jnp.where(isc, c, rv))
            cns.append(jnp.where(lane_ok, m - g_load(k, l), BIG))
            dec[pl.ds((k * H + l) * LANES, LANES)] = jp
        return tuple(cns)

    cfins = lax.fori_loop(1, H, fwd, c0s, unroll=8)

    idx0s = []
    for k in range(NWAY):
        mn = jnp.min(cfins[k])
        idx0 = plsc.all_reduce_ffs(cfins[k] == mn)
        idx0s.append(idx0 + zero16)

    def back(i2, idxvs):
        l = (H - 1) - i2
        nxt = []
        for k in range(NWAY):
            idxv = idxvs[k]
            plsc.store_scatter(pos, [k * H + l + zero16], idxv + r0s[k],
                               mask=lane0)
            nxt.append(plsc.load_gather(dec, [(k * H + l) * LANES + idxv]))
        return tuple(nxt)

    lax.fori_loop(0, H, back, tuple(idx0s), unroll=8)

    for k in range(NWAY):
        @pl.when(pids[k] < NPROB)
        def _():
            pltpu.sync_copy(pos.at[pl.ds(k * H, H)],
                            out_hbm.at[pl.ds(pids[k] * H, H)])


def _paths_sparsecore(g2):
    mesh = plsc.VectorSubcoreMesh(core_axis_name="c", subcore_axis_name="s")
    return pl.kernel(
        _sc_body,
        mesh=mesh,
        compiler_params=pltpu.CompilerParams(needs_layout_passes=False),
        out_type=jax.ShapeDtypeStruct((NPROB * H,), jnp.int32),
        scratch_types=(
            [pltpu.VMEM((BROWS, W), jnp.float32) for _ in range(NWAY)]
            + [pltpu.VMEM((NWAY * H * LANES,), jnp.int32),
               pltpu.VMEM((NWAY * H,), jnp.int32)]
        ),
    )(g2)


def _label_body(pos_ref, out_ref):
    pb = pos_ref[0]
    col = lax.broadcasted_iota(jnp.int32, (H, W), 1)
    row = lax.broadcasted_iota(jnp.int32, (H, W), 0)
    acc = jnp.zeros((H, W), jnp.int32)
    for p in range(NPATH):
        acc = acc + (col >= pb[p][:, None]).astype(jnp.int32)
        acc = acc + 8 * (row >= pb[NPATH + p][None, :]).astype(jnp.int32)
    out_ref[0] = acc


def _labels_pallas(pos):
    return pl.pallas_call(
        _label_body,
        grid=(8,),
        in_specs=[pl.BlockSpec((1, 14, H), lambda b: (b, 0, 0))],
        out_specs=pl.BlockSpec((1, H, W), lambda b: (b, 0, 0)),
        out_shape=jax.ShapeDtypeStruct((8, H, W), jnp.int32),
    )(pos)


def kernel(x):
    g2 = _gradient_map_pallas(x)
    pos = _paths_sparsecore(g2).reshape(8, 14, H)
    return _labels_pallas(pos)


# speedup vs baseline: 1.0334x; 1.0334x over previous
import jax
import jax.numpy as jnp
from jax import lax
from jax.experimental import pallas as pl
from jax.experimental.pallas import tpu as pltpu
from jax.experimental.pallas import tpu_sc as plsc

H = 512
W = 512
NPATH = 7
NB = 11
LANES = 16
NPROB = 8 * 2 * NPATH
BROWS = 24
DOFF = 3
BIG = 1e30


def _gm_body(x_ref, out_ref):
    xb = x_ref[0]
    gray = 0.2989 * xb[0] + 0.587 * xb[1] + 0.114 * xb[2]
    zr = jnp.zeros((1, W), jnp.float32)
    zc = jnp.zeros((H, 1), jnp.float32)
    up = jnp.concatenate([gray[1:], zr], axis=0)
    dn = jnp.concatenate([zr, gray[:-1]], axis=0)
    s = dn + 2.0 * gray + up
    gx = (jnp.concatenate([s[:, 1:], zc], axis=1)
          - jnp.concatenate([zc, s[:, :-1]], axis=1))
    d = (jnp.concatenate([gray[:, 1:], zc], axis=1) + 2.0 * gray
         + jnp.concatenate([zc, gray[:, :-1]], axis=1))
    gy = (jnp.concatenate([d[1:], zr], axis=0)
          - jnp.concatenate([zr, d[:-1]], axis=0))
    gm = jnp.sqrt(gx * gx + gy * gy + 1e-8)
    gmt = gm.T
    for p in range(NPATH):
        c0 = 64 * p + 56
        out_ref[0, 0, p] = gm[c0:c0 + BROWS, :]
        out_ref[0, 1, p] = gmt[c0:c0 + BROWS, :]


def _gradient_map_pallas(x):
    return pl.pallas_call(
        _gm_body,
        grid=(8,),
        in_specs=[pl.BlockSpec((1, 3, H, W), lambda b: (b, 0, 0, 0))],
        out_specs=pl.BlockSpec((1, 2, NPATH, BROWS, W),
                               lambda b: (b, 0, 0, 0, 0)),
        out_shape=jax.ShapeDtypeStruct((8, 2, NPATH, BROWS, W), jnp.float32),
    )(x)


NWAY = 4


def _sc_body(g2_hbm, out_hbm, band0, band1, band2, band3, dec, pos):
    bands = [band0, band1, band2, band3]
    wid = lax.axis_index("s") * 2 + lax.axis_index("c")
    iota = lax.iota(jnp.int32, LANES)
    lane_ok = iota < NB
    iota_d = iota + DOFF
    jl = jnp.maximum(iota - 1, 0)
    jr = jnp.minimum(iota + 1, LANES - 1)
    zero16 = jnp.zeros((LANES,), jnp.int32)
    lane0 = iota == 0

    pids = [wid + 32 * k for k in range(NWAY)]
    r0s = []
    for k in range(NWAY):
        pid = jnp.minimum(pids[k], NPROB - 1)
        b = pid // 14
        t = pid - 14 * b
        orient = jnp.where(t < NPATH, 1, 0)
        p = lax.rem(t, NPATH)
        r0s.append(64 * p + 59)
        pltpu.sync_copy(g2_hbm.at[b, orient, p, :, :], bands[k])

    def g_load(k, l):
        return plsc.load_gather(bands[k], [iota_d, l + zero16])

    c0s = tuple(jnp.where(lane_ok, -g_load(k, 0), BIG) for k in range(NWAY))

    def fwd(l, cs):
        cns = []
        for k in range(NWAY):
            c = cs[k]
            lv = c.at[jl].get(mode="promise_in_bounds")
            rv = c.at[jr].get(mode="promise_in_bounds")
            isl = jnp.logical_and(lv <= c, lv <= rv)
            isc = jnp.logical_and(jnp.logical_not(isl), c <= rv)
            jp = jnp.where(isl, jl, jnp.where(isc, iota, jr))
            m = jnp.where(isl, lv, jnp.where(isc, c, rv))
            cns.append(jnp.where(lane_ok, m - g_load(k, l), BIG))
            dec[pl.ds((k * H + l) * LANES, LANES)] = jp
        return tuple(cns)

    cfins = lax.fori_loop(1, H, fwd, c0s, unroll=4)

    idx0s = []
    for k in range(NWAY):
        mn = jnp.min(cfins[k])
        idx0 = plsc.all_reduce_ffs(cfins[k] == mn)
        idx0s.append(idx0 + zero16)

    def back(i2, idxvs):
        l = (H - 1) - i2
        nxt = []
        for k in range(NWAY):
            idxv = idxvs[k]
            plsc.store_scatter(pos, [k * H + l + zero16], idxv + r0s[k],
                               mask=lane0)
            nxt.append(plsc.load_gather(dec, [(k * H + l) * LANES + idxv]))
        return tuple(nxt)

    lax.fori_loop(0, H, back, tuple(idx0s), unroll=4)

    for k in range(NWAY):
        @pl.when(pids[k] < NPROB)
        def _():
            pltpu.sync_copy(pos.at[pl.ds(k * H, H)],
                            out_hbm.at[pl.ds(pids[k] * H, H)])


def _paths_sparsecore(g2):
    mesh = plsc.VectorSubcoreMesh(core_axis_name="c", subcore_axis_name="s")
    return pl.kernel(
        _sc_body,
        mesh=mesh,
        compiler_params=pltpu.CompilerParams(needs_layout_passes=False),
        out_type=jax.ShapeDtypeStruct((NPROB * H,), jnp.int32),
        scratch_types=(
            [pltpu.VMEM((BROWS, W), jnp.float32) for _ in range(NWAY)]
            + [pltpu.VMEM((NWAY * H * LANES,), jnp.int32),
               pltpu.VMEM((NWAY * H,), jnp.int32)]
        ),
    )(g2)


def _label_body(pos_ref, out_ref):
    pb = pos_ref[0]
    col = lax.broadcasted_iota(jnp.int32, (H, W), 1)
    row = lax.broadcasted_iota(jnp.int32, (H, W), 0)
    acc = jnp.zeros((H, W), jnp.int32)
    for p in range(NPATH):
        acc = acc + (col >= pb[p][:, None]).astype(jnp.int32)
        acc = acc + 8 * (row >= pb[NPATH + p][None, :]).astype(jnp.int32)
    out_ref[0] = acc


def _labels_pallas(pos):
    return pl.pallas_call(
        _label_body,
        grid=(8,),
        in_specs=[pl.BlockSpec((1, 14, H), lambda b: (b, 0, 0))],
        out_specs=pl.BlockSpec((1, H, W), lambda b: (b, 0, 0)),
        out_shape=jax.ShapeDtypeStruct((8, H, W), jnp.int32),
    )(pos)


def kernel(x):
    g2 = _gradient_map_pallas(x)
    pos = _paths_sparsecore(g2).reshape(8, 14, H)
    return _labels_pallas(pos)
